# baseline (speedup 1.0000x reference)
import jax
import jax.numpy as jnp
from jax.experimental import pallas as pl
from jax.experimental.pallas import tpu as pltpu


def _bcast_copy_body(table_ref, out_ref):
    out_ref[...] = jnp.broadcast_to(table_ref[...][None], out_ref.shape)


def kernel(inputs, pos_embedding):
    B, S = inputs.shape
    P, D = pos_embedding.shape
    BS = 2048
    grid = (S // BS, B // 2)
    out = pl.pallas_call(
        _bcast_copy_body,
        grid=grid,
        in_specs=[pl.BlockSpec((BS, D), lambda i, j: (i, 0))],
        out_specs=pl.BlockSpec((2, BS, D), lambda i, j: (j, i, 0)),
        out_shape=jax.ShapeDtypeStruct((B, S, D), pos_embedding.dtype),
        compiler_params=pltpu.CompilerParams(vmem_limit_bytes=63 * 1024 * 1024),
    )(pos_embedding)
    return out
